# baseline (speedup 1.0000x reference)
import functools

import jax
import jax.numpy as jnp
from jax import lax
from jax.experimental import pallas as pl
from jax.experimental.pallas import tpu as pltpu
from jax.experimental.pallas import tpu_sc as plsc

_B = 4
_N = 8192
_M = 1024
_S = 16
_R2 = 0.25
_CIN = 64
_CP = 128
_EPS = 1e-3

_HIGH = jax.lax.Precision.HIGHEST


_NSUB = 64


def _fps_body(pts_ref, cxyz_ref, t1_ref, t2_ref):
    px = [pts_ref[b, 0] for b in range(_B)]
    py = [pts_ref[b, 1] for b in range(_B)]
    pz = [pts_ref[b, 2] for b in range(_B)]
    for b in range(_B):
        t2_ref[b] = px[b] * px[b] + py[b] * py[b] + pz[b] * pz[b]
    sub = lax.broadcasted_iota(jnp.int32, (_NSUB, 128), 0)
    lane = lax.broadcasted_iota(jnp.int32, (_NSUB, 128), 1)
    iota_n = sub * 128 + lane
    msub = lax.broadcasted_iota(jnp.int32, (1, _M // 128, 128), 1)
    mlane = lax.broadcasted_iota(jnp.int32, (1, _M // 128, 128), 2)
    iota_m = msub * 128 + mlane
    iota_m2 = iota_m[0]

    def step(t, carry):
        dists = list(carry[:_B])
        last = list(carry[_B:])
        for b in range(_B):
            eq = iota_n == last[b]
            curx = jnp.sum(jnp.where(eq, px[b], 0.0), axis=(0, 1),
                           keepdims=True)
            cury = jnp.sum(jnp.where(eq, py[b], 0.0), axis=(0, 1),
                           keepdims=True)
            curz = jnp.sum(jnp.where(eq, pz[b], 0.0), axis=(0, 1),
                           keepdims=True)
            cur3 = jnp.concatenate(
                [curx[None], cury[None], curz[None]], axis=0)
            cxyz_ref[b] = jnp.where(iota_m == t, cur3, cxyz_ref[b])
            t1v = curx * curx + cury * cury + curz * curz
            t1_ref[b] = jnp.where(iota_m2 == t, t1v, t1_ref[b])
            dx = px[b] - curx
            dy = py[b] - cury
            dz = pz[b] - curz
            d = dx * dx + dy * dy + dz * dz
            dists[b] = jnp.minimum(dists[b], d)
            mx = jnp.max(dists[b], axis=(0, 1), keepdims=True)
            last[b] = jnp.min(jnp.where(dists[b] == mx, iota_n, _N),
                              axis=(0, 1), keepdims=True).astype(jnp.int32)
        return tuple(dists) + tuple(last)

    def step4(i, carry):
        for k in range(4):
            carry = step(i * 4 + k, carry)
        return carry

    dists0 = tuple(jnp.full((_NSUB, 128), 1e10, dtype=jnp.float32)
                   for _ in range(_B))
    last0 = tuple(jnp.zeros((1, 1), dtype=jnp.int32) for _ in range(_B))
    lax.fori_loop(0, _M // 4, step4, dists0 + last0)


def _fps(points):
    cxyz, t1, t2 = pl.pallas_call(
        _fps_body,
        out_shape=[
            jax.ShapeDtypeStruct((_B, 3, _M // 128, 128), jnp.float32),
            jax.ShapeDtypeStruct((_B, _M // 128, 128), jnp.float32),
            jax.ShapeDtypeStruct((_B, _NSUB, 128), jnp.float32),
        ],
    )(points.reshape(_B, 3, _NSUB, 128))
    return cxyz.reshape(_B, 3, _M), t1.reshape(_B, 1, _M), t2.reshape(_B, _N)


def _prep_body(pts_ref, feat_ref, tab_ref):
    xt = jnp.transpose(pts_ref[0], (1, 0))
    ft = jnp.transpose(feat_ref[0], (1, 0))
    tab_ref[...] = jnp.concatenate(
        [xt, ft, jnp.zeros((_N, _CP - 3 - _CIN), jnp.float32)], axis=1)


def _prep(points, features):
    return pl.pallas_call(
        _prep_body,
        grid=(_B,),
        in_specs=[
            pl.BlockSpec((1, 3, _N), lambda b: (b, 0, 0)),
            pl.BlockSpec((1, _CIN, _N), lambda b: (b, 0, 0)),
        ],
        out_specs=pl.BlockSpec((_N, _CP), lambda b: (b, 0)),
        out_shape=jax.ShapeDtypeStruct((_B * _N, _CP), jnp.float32),
    )(points, features)


_MBLK = 128


def _ballq_body(xt_ref, c_ref, t1_ref, t2_ref, idx_ref):
    b = pl.program_id(0)
    xt = xt_ref[:, 0:8]
    ca = c_ref[0]
    t1 = t1_ref[0]
    t2 = t2_ref[0]
    e = jnp.dot(xt, ca, preferred_element_type=jnp.float32)
    d2 = t1 + t2 - 2.0 * e
    ri = lax.broadcasted_iota(jnp.int32, (_N, 1), 0)
    bitv = jnp.left_shift(1, jnp.bitwise_and(ri, 15))
    bits = jnp.where(d2 < _R2, jnp.broadcast_to(bitv, (_N, _MBLK)), 0)
    w = jnp.sum(bits.reshape(_N // 16, 16, _MBLK), axis=1)
    g_iota = lax.broadcasted_iota(jnp.int32, (_N // 16, _MBLK), 0)
    base_g = g_iota * 16
    sels = []
    for _ in range(_S):
        isol = jnp.bitwise_and(w, -w)
        f = isol.astype(jnp.float32)
        ebit = jnp.right_shift(
            lax.bitcast_convert_type(f, jnp.int32), 23) - 127
        key = jnp.where(w != 0, base_g + ebit, _N)
        mk = jnp.min(key, axis=0, keepdims=True)
        sels.append(mk)
        gsel = jnp.right_shift(mk, 4)
        w = jnp.where(g_iota == gsel,
                      jnp.bitwise_and(w, w - 1), w)
    sel = jnp.concatenate(sels, axis=0)
    first = sel[0:1, :]
    sel = jnp.where(sel >= _N, jnp.broadcast_to(first, sel.shape), sel)
    sel = jnp.where(sel >= _N, 0, sel)
    idx_ref[0] = sel + b * _N


def _ballq(table, c8, t1, t2):
    return pl.pallas_call(
        _ballq_body,
        grid=(_B, _M // _MBLK),
        in_specs=[
            pl.BlockSpec((_N, _CP), lambda b, mi: (b, 0)),
            pl.BlockSpec((1, 8, _MBLK), lambda b, mi: (b, 0, mi)),
            pl.BlockSpec((1, 1, _MBLK), lambda b, mi: (b, 0, mi)),
            pl.BlockSpec((1, _N, 1), lambda b, mi: (b, 0, 0)),
        ],
        out_specs=pl.BlockSpec((1, _S, _MBLK), lambda b, mi: (b, 0, mi)),
        out_shape=jax.ShapeDtypeStruct((_B, _S, _M), jnp.int32),
        compiler_params=pltpu.CompilerParams(vmem_limit_bytes=100_000_000),
    )(table, c8, t1, t2)


_NW = 32
_ROWS = _B * _M * _S
_PW = _ROWS // _NW
_CHUNK = 256
_NBUF = 3


def _gather_rows(idx_flat, table):
    mesh = plsc.VectorSubcoreMesh(core_axis_name="c", subcore_axis_name="s")
    nch = _PW // _CHUNK

    @functools.partial(
        pl.kernel,
        out_type=jax.ShapeDtypeStruct((_ROWS, _CP), jnp.float32),
        mesh=mesh,
        scratch_types=[
            pltpu.VMEM((_PW,), jnp.int32),
            pltpu.VMEM((_NBUF, _CHUNK, _CP), jnp.float32),
            pltpu.SemaphoreType.DMA((_NBUF,)),
            pltpu.SemaphoreType.DMA((_NBUF,)),
        ],
    )
    def k(idx_hbm, tab_hbm, out_hbm, idx_v, rows_v, sg, sw):
        wid = lax.axis_index("s") * 2 + lax.axis_index("c")
        base = wid * _PW
        pltpu.sync_copy(idx_hbm.at[pl.ds(base, _PW)], idx_v)
        cg = [None] * nch
        cw = [None] * nch
        for j in range(min(_NBUF, nch)):
            cg[j] = pltpu.async_copy(
                tab_hbm.at[idx_v.at[pl.ds(j * _CHUNK, _CHUNK)]],
                rows_v.at[j], sg.at[j])
        for j in range(nch):
            bi = j % _NBUF
            cg[j].wait()
            cw[j] = pltpu.async_copy(
                rows_v.at[bi], out_hbm.at[pl.ds(base + j * _CHUNK, _CHUNK)],
                sw.at[bi])
            nj = j + _NBUF
            if nj < nch:
                cw[j].wait()
                cg[nj] = pltpu.async_copy(
                    tab_hbm.at[idx_v.at[pl.ds(nj * _CHUNK, _CHUNK)]],
                    rows_v.at[bi], sg.at[bi])
        for j in range(max(nch - _NBUF, 0), nch):
            cw[j].wait()

    return k(idx_flat, table)


_RBLK = 2048
_NBLK = _ROWS // _RBLK


def _mm0_body(xg_ref, cp_ref, w_ref, b_ref, y_ref, s_ref, ss_ref):
    i = pl.program_id(0)

    @pl.when(i == 0)
    def _():
        s_ref[...] = jnp.zeros_like(s_ref)
        ss_ref[...] = jnp.zeros_like(ss_ref)

    c = cp_ref[...]
    cexp = jnp.broadcast_to(c[:, None, :], (_RBLK // _S, _S, _CP))
    x = xg_ref[...] - cexp.reshape(_RBLK, _CP)
    y = jnp.dot(x, w_ref[...], preferred_element_type=jnp.float32,
                precision=_HIGH) + b_ref[...]
    y_ref[...] = y
    s_ref[...] += jnp.sum(y, axis=0, keepdims=True)
    ss_ref[...] += jnp.sum(y * y, axis=0, keepdims=True)


def _mm0(xg, cpad, w0p, b0r):
    return pl.pallas_call(
        _mm0_body,
        grid=(_NBLK,),
        in_specs=[
            pl.BlockSpec((_RBLK, _CP), lambda i: (i, 0)),
            pl.BlockSpec((_RBLK // _S, _CP), lambda i: (i, 0)),
            pl.BlockSpec((_CP, 64), lambda i: (0, 0)),
            pl.BlockSpec((1, 64), lambda i: (0, 0)),
        ],
        out_specs=[
            pl.BlockSpec((_RBLK, 64), lambda i: (i, 0)),
            pl.BlockSpec((1, 64), lambda i: (0, 0)),
            pl.BlockSpec((1, 64), lambda i: (0, 0)),
        ],
        out_shape=[
            jax.ShapeDtypeStruct((_ROWS, 64), jnp.float32),
            jax.ShapeDtypeStruct((1, 64), jnp.float32),
            jax.ShapeDtypeStruct((1, 64), jnp.float32),
        ],
    )(xg, cpad, w0p, b0r)


def _bnmm_body(y_ref, s_ref, ss_ref, g_ref, be_ref, w_ref, b_ref,
               o_ref, so_ref, sso_ref):
    i = pl.program_id(0)

    @pl.when(i == 0)
    def _():
        so_ref[...] = jnp.zeros_like(so_ref)
        sso_ref[...] = jnp.zeros_like(sso_ref)

    inv_n = 1.0 / float(_ROWS)
    mean = s_ref[...] * inv_n
    var = ss_ref[...] * inv_n - mean * mean
    rstd = 1.0 / jnp.sqrt(var + _EPS)
    z = (y_ref[...] - mean) * rstd * g_ref[...] + be_ref[...]
    z = jnp.maximum(z, 0.0)
    o = jnp.dot(z, w_ref[...], preferred_element_type=jnp.float32,
                precision=_HIGH) + b_ref[...]
    o_ref[...] = o
    so_ref[...] += jnp.sum(o, axis=0, keepdims=True)
    sso_ref[...] += jnp.sum(o * o, axis=0, keepdims=True)


def _bnmm(y, s, ss, g, be, wt, br, cout):
    cin = y.shape[1]
    return pl.pallas_call(
        _bnmm_body,
        grid=(_NBLK,),
        in_specs=[
            pl.BlockSpec((_RBLK, cin), lambda i: (i, 0)),
            pl.BlockSpec((1, cin), lambda i: (0, 0)),
            pl.BlockSpec((1, cin), lambda i: (0, 0)),
            pl.BlockSpec((1, cin), lambda i: (0, 0)),
            pl.BlockSpec((1, cin), lambda i: (0, 0)),
            pl.BlockSpec((cin, cout), lambda i: (0, 0)),
            pl.BlockSpec((1, cout), lambda i: (0, 0)),
        ],
        out_specs=[
            pl.BlockSpec((_RBLK, cout), lambda i: (i, 0)),
            pl.BlockSpec((1, cout), lambda i: (0, 0)),
            pl.BlockSpec((1, cout), lambda i: (0, 0)),
        ],
        out_shape=[
            jax.ShapeDtypeStruct((_ROWS, cout), jnp.float32),
            jax.ShapeDtypeStruct((1, cout), jnp.float32),
            jax.ShapeDtypeStruct((1, cout), jnp.float32),
        ],
    )(y, s, ss, g, be, wt, br)


def _bnpool_body(y_ref, s_ref, ss_ref, g_ref, be_ref, o_ref):
    inv_n = 1.0 / float(_ROWS)
    mean = s_ref[...] * inv_n
    var = ss_ref[...] * inv_n - mean * mean
    rstd = 1.0 / jnp.sqrt(var + _EPS)
    z = (y_ref[...] - mean) * rstd * g_ref[...] + be_ref[...]
    z = jnp.maximum(z, 0.0)
    o_ref[...] = jnp.max(z.reshape(_RBLK // _S, _S, 128), axis=1)


def _bnpool(y, s, ss, g, be):
    return pl.pallas_call(
        _bnpool_body,
        grid=(_NBLK,),
        in_specs=[
            pl.BlockSpec((_RBLK, 128), lambda i: (i, 0)),
            pl.BlockSpec((1, 128), lambda i: (0, 0)),
            pl.BlockSpec((1, 128), lambda i: (0, 0)),
            pl.BlockSpec((1, 128), lambda i: (0, 0)),
            pl.BlockSpec((1, 128), lambda i: (0, 0)),
        ],
        out_specs=pl.BlockSpec((_RBLK // _S, 128), lambda i: (i, 0)),
        out_shape=jax.ShapeDtypeStruct((_B * _M, 128), jnp.float32),
    )(y, s, ss, g, be)


def kernel(points, features, W0, b0, gamma0, beta0, W1, b1, gamma1, beta1,
           W2, b2, gamma2, beta2):
    cxyz, t1, t2 = _fps(points)

    table = _prep(points, features)
    c8 = jnp.pad(cxyz, ((0, 0), (0, 5), (0, 0)))
    idx_t = _ballq(table, c8, t1, t2[:, :, None])
    idx = jnp.transpose(idx_t, (0, 2, 1))

    xg = _gather_rows(idx.reshape(_ROWS), table)

    crows = jnp.transpose(cxyz, (0, 2, 1)).reshape(_B * _M, 3)
    cpad = jnp.pad(crows, ((0, 0), (0, _CP - 3)))

    w0p = jnp.pad(W0.T, ((0, _CP - W0.shape[1]), (0, 0)))
    y0, s0, ss0 = _mm0(xg, cpad, w0p, b0.reshape(1, -1))
    y1, s1, ss1 = _bnmm(y0, s0, ss0, gamma0.reshape(1, -1),
                        beta0.reshape(1, -1), W1.T, b1.reshape(1, -1), 64)
    y2, s2, ss2 = _bnmm(y1, s1, ss1, gamma1.reshape(1, -1),
                        beta1.reshape(1, -1), W2.T, b2.reshape(1, -1), 128)
    pooled = _bnpool(y2, s2, ss2, gamma2.reshape(1, -1), beta2.reshape(1, -1))

    new_features = jnp.transpose(pooled.reshape(_B, _M, 128), (0, 2, 1))
    return (cxyz, new_features)


# speedup vs baseline: 2.1135x; 2.1135x over previous
import functools

import jax
import jax.numpy as jnp
from jax import lax
from jax.experimental import pallas as pl
from jax.experimental.pallas import tpu as pltpu
from jax.experimental.pallas import tpu_sc as plsc

_B = 4
_N = 8192
_M = 1024
_S = 16
_R2 = 0.25
_CIN = 64
_CP = 128
_EPS = 1e-3

_HIGH = jax.lax.Precision.HIGHEST


_NSUB = 64


def _fps_body(pts_ref, cxyz_ref, t1_ref, t2_ref):
    px = pts_ref[:, 0]
    py = pts_ref[:, 1]
    pz = pts_ref[:, 2]
    t2_ref[...] = px * px + py * py + pz * pz
    sub = lax.broadcasted_iota(jnp.int32, (1, _NSUB, 128), 1)
    lane = lax.broadcasted_iota(jnp.int32, (1, _NSUB, 128), 2)
    iota_n = sub * 128 + lane
    msub = lax.broadcasted_iota(jnp.int32, (1, 1, _M // 128, 128), 2)
    mlane = lax.broadcasted_iota(jnp.int32, (1, 1, _M // 128, 128), 3)
    iota_m = msub * 128 + mlane
    iota_m2 = iota_m[:, 0]

    def step(t, carry):
        dists, last = carry
        eq = iota_n == last
        curx = jnp.sum(jnp.where(eq, px, 0.0), axis=(1, 2), keepdims=True)
        cury = jnp.sum(jnp.where(eq, py, 0.0), axis=(1, 2), keepdims=True)
        curz = jnp.sum(jnp.where(eq, pz, 0.0), axis=(1, 2), keepdims=True)
        cur3 = jnp.concatenate(
            [curx[:, None], cury[:, None], curz[:, None]], axis=1)
        cxyz_ref[...] = jnp.where(iota_m == t, cur3, cxyz_ref[...])
        t1v = curx * curx + cury * cury + curz * curz
        t1_ref[...] = jnp.where(iota_m2 == t, t1v, t1_ref[...])
        dx = px - curx
        dy = py - cury
        dz = pz - curz
        d = dx * dx + dy * dy + dz * dz
        dists = jnp.minimum(dists, d)
        mx = jnp.max(dists, axis=(1, 2), keepdims=True)
        nxt = jnp.min(jnp.where(dists == mx, iota_n, _N), axis=(1, 2),
                      keepdims=True).astype(jnp.int32)
        return dists, nxt

    def step4(i, carry):
        for k in range(4):
            carry = step(i * 4 + k, carry)
        return carry

    dists0 = jnp.full((_B, _NSUB, 128), 1e10, dtype=jnp.float32)
    last0 = jnp.zeros((_B, 1, 1), dtype=jnp.int32)
    lax.fori_loop(0, _M // 4, step4, (dists0, last0))


def _fps(points):
    cxyz, t1, t2 = pl.pallas_call(
        _fps_body,
        out_shape=[
            jax.ShapeDtypeStruct((_B, 3, _M // 128, 128), jnp.float32),
            jax.ShapeDtypeStruct((_B, _M // 128, 128), jnp.float32),
            jax.ShapeDtypeStruct((_B, _NSUB, 128), jnp.float32),
        ],
    )(points.reshape(_B, 3, _NSUB, 128))
    return cxyz.reshape(_B, 3, _M), t1.reshape(_B, 1, _M), t2.reshape(_B, _N)


def _prep_body(pts_ref, feat_ref, tab_ref):
    xt = jnp.transpose(pts_ref[0], (1, 0))
    ft = jnp.transpose(feat_ref[0], (1, 0))
    tab_ref[...] = jnp.concatenate(
        [xt, ft, jnp.zeros((_N, _CP - 3 - _CIN), jnp.float32)], axis=1)


def _prep(points, features):
    return pl.pallas_call(
        _prep_body,
        grid=(_B,),
        in_specs=[
            pl.BlockSpec((1, 3, _N), lambda b: (b, 0, 0)),
            pl.BlockSpec((1, _CIN, _N), lambda b: (b, 0, 0)),
        ],
        out_specs=pl.BlockSpec((_N, _CP), lambda b: (b, 0)),
        out_shape=jax.ShapeDtypeStruct((_B * _N, _CP), jnp.float32),
    )(points, features)


_MBLK = 128


def _ballq_body(xt_ref, c_ref, t1_ref, t2_ref, idx_ref):
    b = pl.program_id(0)
    xt = xt_ref[:, 0:8]
    ca = c_ref[0]
    t1 = t1_ref[0]
    t2 = t2_ref[0]
    e = jnp.dot(xt, ca, preferred_element_type=jnp.float32)
    d2 = t1 + t2 - 2.0 * e
    ri = lax.broadcasted_iota(jnp.int32, (_N, 1), 0)
    bitv = jnp.left_shift(1, jnp.bitwise_and(ri, 15))
    bits = jnp.where(d2 < _R2, jnp.broadcast_to(bitv, (_N, _MBLK)), 0)
    w = jnp.sum(bits.reshape(_N // 16, 16, _MBLK), axis=1)
    g_iota = lax.broadcasted_iota(jnp.int32, (_N // 16, _MBLK), 0)
    base_g = g_iota * 16
    sels = []
    for _ in range(_S):
        isol = jnp.bitwise_and(w, -w)
        f = isol.astype(jnp.float32)
        ebit = jnp.right_shift(
            lax.bitcast_convert_type(f, jnp.int32), 23) - 127
        key = jnp.where(w != 0, base_g + ebit, _N)
        mk = jnp.min(key, axis=0, keepdims=True)
        sels.append(mk)
        gsel = jnp.right_shift(mk, 4)
        w = jnp.where(g_iota == gsel,
                      jnp.bitwise_and(w, w - 1), w)
    sel = jnp.concatenate(sels, axis=0)
    first = sel[0:1, :]
    sel = jnp.where(sel >= _N, jnp.broadcast_to(first, sel.shape), sel)
    sel = jnp.where(sel >= _N, 0, sel)
    idx_ref[0] = sel + b * _N


def _ballq(table, c8, t1, t2):
    return pl.pallas_call(
        _ballq_body,
        grid=(_B, _M // _MBLK),
        in_specs=[
            pl.BlockSpec((_N, _CP), lambda b, mi: (b, 0)),
            pl.BlockSpec((1, 8, _MBLK), lambda b, mi: (b, 0, mi)),
            pl.BlockSpec((1, 1, _MBLK), lambda b, mi: (b, 0, mi)),
            pl.BlockSpec((1, _N, 1), lambda b, mi: (b, 0, 0)),
        ],
        out_specs=pl.BlockSpec((1, _S, _MBLK), lambda b, mi: (b, 0, mi)),
        out_shape=jax.ShapeDtypeStruct((_B, _S, _M), jnp.int32),
        compiler_params=pltpu.CompilerParams(vmem_limit_bytes=100_000_000),
    )(table, c8, t1, t2)


_NW = 32
_ROWS = _B * _M * _S
_PW = _ROWS // _NW
_CHUNK = 256
_NBUF = 3


def _gather_rows(idx_flat, table):
    mesh = plsc.VectorSubcoreMesh(core_axis_name="c", subcore_axis_name="s")
    nch = _PW // _CHUNK

    @functools.partial(
        pl.kernel,
        out_type=jax.ShapeDtypeStruct((_ROWS, _CP), jnp.float32),
        mesh=mesh,
        scratch_types=[
            pltpu.VMEM((_PW,), jnp.int32),
            pltpu.VMEM((_NBUF, _CHUNK, _CP), jnp.float32),
            pltpu.SemaphoreType.DMA((_NBUF,)),
            pltpu.SemaphoreType.DMA((_NBUF,)),
        ],
    )
    def k(idx_hbm, tab_hbm, out_hbm, idx_v, rows_v, sg, sw):
        wid = lax.axis_index("s") * 2 + lax.axis_index("c")
        base = wid * _PW
        pltpu.sync_copy(idx_hbm.at[pl.ds(base, _PW)], idx_v)
        cg = [None] * nch
        cw = [None] * nch
        for j in range(min(_NBUF, nch)):
            cg[j] = pltpu.async_copy(
                tab_hbm.at[idx_v.at[pl.ds(j * _CHUNK, _CHUNK)]],
                rows_v.at[j], sg.at[j])
        for j in range(nch):
            bi = j % _NBUF
            cg[j].wait()
            cw[j] = pltpu.async_copy(
                rows_v.at[bi], out_hbm.at[pl.ds(base + j * _CHUNK, _CHUNK)],
                sw.at[bi])
            nj = j + _NBUF
            if nj < nch:
                cw[j].wait()
                cg[nj] = pltpu.async_copy(
                    tab_hbm.at[idx_v.at[pl.ds(nj * _CHUNK, _CHUNK)]],
                    rows_v.at[bi], sg.at[bi])
        for j in range(max(nch - _NBUF, 0), nch):
            cw[j].wait()

    return k(idx_flat, table)


_RBLK = 2048
_NBLK = _ROWS // _RBLK


def _mm0_body(xg_ref, cp_ref, w_ref, b_ref, y_ref, s_ref, ss_ref):
    i = pl.program_id(0)

    @pl.when(i == 0)
    def _():
        s_ref[...] = jnp.zeros_like(s_ref)
        ss_ref[...] = jnp.zeros_like(ss_ref)

    c = cp_ref[...]
    cexp = jnp.broadcast_to(c[:, None, :], (_RBLK // _S, _S, _CP))
    x = xg_ref[...] - cexp.reshape(_RBLK, _CP)
    y = jnp.dot(x, w_ref[...], preferred_element_type=jnp.float32,
                precision=_HIGH) + b_ref[...]
    y_ref[...] = y
    s_ref[...] += jnp.sum(y, axis=0, keepdims=True)
    ss_ref[...] += jnp.sum(y * y, axis=0, keepdims=True)


def _mm0(xg, cpad, w0p, b0r):
    return pl.pallas_call(
        _mm0_body,
        grid=(_NBLK,),
        in_specs=[
            pl.BlockSpec((_RBLK, _CP), lambda i: (i, 0)),
            pl.BlockSpec((_RBLK // _S, _CP), lambda i: (i, 0)),
            pl.BlockSpec((_CP, 64), lambda i: (0, 0)),
            pl.BlockSpec((1, 64), lambda i: (0, 0)),
        ],
        out_specs=[
            pl.BlockSpec((_RBLK, 64), lambda i: (i, 0)),
            pl.BlockSpec((1, 64), lambda i: (0, 0)),
            pl.BlockSpec((1, 64), lambda i: (0, 0)),
        ],
        out_shape=[
            jax.ShapeDtypeStruct((_ROWS, 64), jnp.float32),
            jax.ShapeDtypeStruct((1, 64), jnp.float32),
            jax.ShapeDtypeStruct((1, 64), jnp.float32),
        ],
    )(xg, cpad, w0p, b0r)


def _bnmm_body(y_ref, s_ref, ss_ref, g_ref, be_ref, w_ref, b_ref,
               o_ref, so_ref, sso_ref):
    i = pl.program_id(0)

    @pl.when(i == 0)
    def _():
        so_ref[...] = jnp.zeros_like(so_ref)
        sso_ref[...] = jnp.zeros_like(sso_ref)

    inv_n = 1.0 / float(_ROWS)
    mean = s_ref[...] * inv_n
    var = ss_ref[...] * inv_n - mean * mean
    rstd = 1.0 / jnp.sqrt(var + _EPS)
    z = (y_ref[...] - mean) * rstd * g_ref[...] + be_ref[...]
    z = jnp.maximum(z, 0.0)
    o = jnp.dot(z, w_ref[...], preferred_element_type=jnp.float32,
                precision=_HIGH) + b_ref[...]
    o_ref[...] = o
    so_ref[...] += jnp.sum(o, axis=0, keepdims=True)
    sso_ref[...] += jnp.sum(o * o, axis=0, keepdims=True)


def _bnmm(y, s, ss, g, be, wt, br, cout):
    cin = y.shape[1]
    return pl.pallas_call(
        _bnmm_body,
        grid=(_NBLK,),
        in_specs=[
            pl.BlockSpec((_RBLK, cin), lambda i: (i, 0)),
            pl.BlockSpec((1, cin), lambda i: (0, 0)),
            pl.BlockSpec((1, cin), lambda i: (0, 0)),
            pl.BlockSpec((1, cin), lambda i: (0, 0)),
            pl.BlockSpec((1, cin), lambda i: (0, 0)),
            pl.BlockSpec((cin, cout), lambda i: (0, 0)),
            pl.BlockSpec((1, cout), lambda i: (0, 0)),
        ],
        out_specs=[
            pl.BlockSpec((_RBLK, cout), lambda i: (i, 0)),
            pl.BlockSpec((1, cout), lambda i: (0, 0)),
            pl.BlockSpec((1, cout), lambda i: (0, 0)),
        ],
        out_shape=[
            jax.ShapeDtypeStruct((_ROWS, cout), jnp.float32),
            jax.ShapeDtypeStruct((1, cout), jnp.float32),
            jax.ShapeDtypeStruct((1, cout), jnp.float32),
        ],
    )(y, s, ss, g, be, wt, br)


def _bnpool_body(y_ref, s_ref, ss_ref, g_ref, be_ref, o_ref):
    inv_n = 1.0 / float(_ROWS)
    mean = s_ref[...] * inv_n
    var = ss_ref[...] * inv_n - mean * mean
    rstd = 1.0 / jnp.sqrt(var + _EPS)
    z = (y_ref[...] - mean) * rstd * g_ref[...] + be_ref[...]
    z = jnp.maximum(z, 0.0)
    o_ref[...] = jnp.max(z.reshape(_RBLK // _S, _S, 128), axis=1)


def _bnpool(y, s, ss, g, be):
    return pl.pallas_call(
        _bnpool_body,
        grid=(_NBLK,),
        in_specs=[
            pl.BlockSpec((_RBLK, 128), lambda i: (i, 0)),
            pl.BlockSpec((1, 128), lambda i: (0, 0)),
            pl.BlockSpec((1, 128), lambda i: (0, 0)),
            pl.BlockSpec((1, 128), lambda i: (0, 0)),
            pl.BlockSpec((1, 128), lambda i: (0, 0)),
        ],
        out_specs=pl.BlockSpec((_RBLK // _S, 128), lambda i: (i, 0)),
        out_shape=jax.ShapeDtypeStruct((_B * _M, 128), jnp.float32),
    )(y, s, ss, g, be)


def kernel(points, features, W0, b0, gamma0, beta0, W1, b1, gamma1, beta1,
           W2, b2, gamma2, beta2):
    cxyz, t1, t2 = _fps(points)

    table = _prep(points, features)
    c8 = jnp.pad(cxyz, ((0, 0), (0, 5), (0, 0)))
    idx_t = _ballq(table, c8, t1, t2[:, :, None])
    idx = jnp.transpose(idx_t, (0, 2, 1))

    xg = _gather_rows(idx.reshape(_ROWS), table)

    crows = jnp.transpose(cxyz, (0, 2, 1)).reshape(_B * _M, 3)
    cpad = jnp.pad(crows, ((0, 0), (0, _CP - 3)))

    w0p = jnp.pad(W0.T, ((0, _CP - W0.shape[1]), (0, 0)))
    y0, s0, ss0 = _mm0(xg, cpad, w0p, b0.reshape(1, -1))
    y1, s1, ss1 = _bnmm(y0, s0, ss0, gamma0.reshape(1, -1),
                        beta0.reshape(1, -1), W1.T, b1.reshape(1, -1), 64)
    y2, s2, ss2 = _bnmm(y1, s1, ss1, gamma1.reshape(1, -1),
                        beta1.reshape(1, -1), W2.T, b2.reshape(1, -1), 128)
    pooled = _bnpool(y2, s2, ss2, gamma2.reshape(1, -1), beta2.reshape(1, -1))

    new_features = jnp.transpose(pooled.reshape(_B, _M, 128), (0, 2, 1))
    return (cxyz, new_features)


# speedup vs baseline: 2.1370x; 1.0111x over previous
import functools

import jax
import jax.numpy as jnp
from jax import lax
from jax.experimental import pallas as pl
from jax.experimental.pallas import tpu as pltpu
from jax.experimental.pallas import tpu_sc as plsc

_B = 4
_N = 8192
_M = 1024
_S = 16
_R2 = 0.25
_CIN = 64
_CP = 128
_EPS = 1e-3

_HIGH = jax.lax.Precision.HIGHEST


_NSUB = 64


def _fps_body(pts_ref, cxyz_ref, t1_ref, t2_ref):
    px = pts_ref[:, 0]
    py = pts_ref[:, 1]
    pz = pts_ref[:, 2]
    t2_ref[...] = px * px + py * py + pz * pz
    sub = lax.broadcasted_iota(jnp.int32, (1, _NSUB, 128), 1)
    lane = lax.broadcasted_iota(jnp.int32, (1, _NSUB, 128), 2)
    iota_n = sub * 128 + lane
    msub = lax.broadcasted_iota(jnp.int32, (1, 1, _M // 128, 128), 2)
    mlane = lax.broadcasted_iota(jnp.int32, (1, 1, _M // 128, 128), 3)
    iota_m = msub * 128 + mlane
    iota_m2 = iota_m[:, 0]

    def step(t, carry):
        dists, last = carry
        eq = iota_n == last
        curx = jnp.sum(jnp.where(eq, px, 0.0), axis=(1, 2), keepdims=True)
        cury = jnp.sum(jnp.where(eq, py, 0.0), axis=(1, 2), keepdims=True)
        curz = jnp.sum(jnp.where(eq, pz, 0.0), axis=(1, 2), keepdims=True)
        cur3 = jnp.concatenate(
            [curx[:, None], cury[:, None], curz[:, None]], axis=1)
        cxyz_ref[...] = jnp.where(iota_m == t, cur3, cxyz_ref[...])
        t1v = curx * curx + cury * cury + curz * curz
        t1_ref[...] = jnp.where(iota_m2 == t, t1v, t1_ref[...])
        dx = px - curx
        dy = py - cury
        dz = pz - curz
        d = dx * dx + dy * dy + dz * dz
        dists = jnp.minimum(dists, d)
        mx = jnp.max(dists, axis=(1, 2), keepdims=True)
        nxt = jnp.min(jnp.where(dists == mx, iota_n, _N), axis=(1, 2),
                      keepdims=True).astype(jnp.int32)
        return dists, nxt

    def step4(i, carry):
        for k in range(8):
            carry = step(i * 8 + k, carry)
        return carry

    dists0 = jnp.full((_B, _NSUB, 128), 1e10, dtype=jnp.float32)
    last0 = jnp.zeros((_B, 1, 1), dtype=jnp.int32)
    lax.fori_loop(0, _M // 8, step4, (dists0, last0))


def _fps(points):
    cxyz, t1, t2 = pl.pallas_call(
        _fps_body,
        out_shape=[
            jax.ShapeDtypeStruct((_B, 3, _M // 128, 128), jnp.float32),
            jax.ShapeDtypeStruct((_B, _M // 128, 128), jnp.float32),
            jax.ShapeDtypeStruct((_B, _NSUB, 128), jnp.float32),
        ],
    )(points.reshape(_B, 3, _NSUB, 128))
    return cxyz.reshape(_B, 3, _M), t1.reshape(_B, 1, _M), t2.reshape(_B, _N)


def _prep_body(pts_ref, feat_ref, tab_ref):
    xt = jnp.transpose(pts_ref[0], (1, 0))
    ft = jnp.transpose(feat_ref[0], (1, 0))
    tab_ref[...] = jnp.concatenate(
        [xt, ft, jnp.zeros((_N, _CP - 3 - _CIN), jnp.float32)], axis=1)


def _prep(points, features):
    return pl.pallas_call(
        _prep_body,
        grid=(_B,),
        in_specs=[
            pl.BlockSpec((1, 3, _N), lambda b: (b, 0, 0)),
            pl.BlockSpec((1, _CIN, _N), lambda b: (b, 0, 0)),
        ],
        out_specs=pl.BlockSpec((_N, _CP), lambda b: (b, 0)),
        out_shape=jax.ShapeDtypeStruct((_B * _N, _CP), jnp.float32),
    )(points, features)


_MBLK = 256


def _ballq_body(xt_ref, c_ref, t1_ref, t2_ref, idx_ref):
    b = pl.program_id(0)
    xt = xt_ref[:, 0:8]
    ca = c_ref[0]
    t1 = t1_ref[0]
    t2 = t2_ref[0]
    e = jnp.dot(xt, ca, preferred_element_type=jnp.float32)
    d2 = t1 + t2 - 2.0 * e
    ri = lax.broadcasted_iota(jnp.int32, (_N, 1), 0)
    bitv = jnp.left_shift(1, jnp.bitwise_and(ri, 15))
    bits = jnp.where(d2 < _R2, jnp.broadcast_to(bitv, (_N, _MBLK)), 0)
    w = jnp.sum(bits.reshape(_N // 16, 16, _MBLK), axis=1)
    g_iota = lax.broadcasted_iota(jnp.int32, (_N // 16, _MBLK), 0)
    base_g = g_iota * 16
    sels = []
    for _ in range(_S):
        isol = jnp.bitwise_and(w, -w)
        f = isol.astype(jnp.float32)
        ebit = jnp.right_shift(
            lax.bitcast_convert_type(f, jnp.int32), 23) - 127
        key = jnp.where(w != 0, base_g + ebit, _N)
        mk = jnp.min(key, axis=0, keepdims=True)
        sels.append(mk)
        gsel = jnp.right_shift(mk, 4)
        w = jnp.where(g_iota == gsel,
                      jnp.bitwise_and(w, w - 1), w)
    sel = jnp.concatenate(sels, axis=0)
    first = sel[0:1, :]
    sel = jnp.where(sel >= _N, jnp.broadcast_to(first, sel.shape), sel)
    sel = jnp.where(sel >= _N, 0, sel)
    idx_ref[0] = sel + b * _N


def _ballq(table, c8, t1, t2):
    return pl.pallas_call(
        _ballq_body,
        grid=(_B, _M // _MBLK),
        in_specs=[
            pl.BlockSpec((_N, _CP), lambda b, mi: (b, 0)),
            pl.BlockSpec((1, 8, _MBLK), lambda b, mi: (b, 0, mi)),
            pl.BlockSpec((1, 1, _MBLK), lambda b, mi: (b, 0, mi)),
            pl.BlockSpec((1, _N, 1), lambda b, mi: (b, 0, 0)),
        ],
        out_specs=pl.BlockSpec((1, _S, _MBLK), lambda b, mi: (b, 0, mi)),
        out_shape=jax.ShapeDtypeStruct((_B, _S, _M), jnp.int32),
        compiler_params=pltpu.CompilerParams(vmem_limit_bytes=100_000_000),
    )(table, c8, t1, t2)


_NW = 32
_ROWS = _B * _M * _S
_PW = _ROWS // _NW
_CHUNK = 256
_NBUF = 3


def _gather_rows(idx_flat, table):
    mesh = plsc.VectorSubcoreMesh(core_axis_name="c", subcore_axis_name="s")
    nch = _PW // _CHUNK

    @functools.partial(
        pl.kernel,
        out_type=jax.ShapeDtypeStruct((_ROWS, _CP), jnp.float32),
        mesh=mesh,
        scratch_types=[
            pltpu.VMEM((_PW,), jnp.int32),
            pltpu.VMEM((_NBUF, _CHUNK, _CP), jnp.float32),
            pltpu.SemaphoreType.DMA((_NBUF,)),
            pltpu.SemaphoreType.DMA((_NBUF,)),
        ],
    )
    def k(idx_hbm, tab_hbm, out_hbm, idx_v, rows_v, sg, sw):
        wid = lax.axis_index("s") * 2 + lax.axis_index("c")
        base = wid * _PW
        pltpu.sync_copy(idx_hbm.at[pl.ds(base, _PW)], idx_v)
        cg = [None] * nch
        cw = [None] * nch
        for j in range(min(_NBUF, nch)):
            cg[j] = pltpu.async_copy(
                tab_hbm.at[idx_v.at[pl.ds(j * _CHUNK, _CHUNK)]],
                rows_v.at[j], sg.at[j])
        for j in range(nch):
            bi = j % _NBUF
            cg[j].wait()
            cw[j] = pltpu.async_copy(
                rows_v.at[bi], out_hbm.at[pl.ds(base + j * _CHUNK, _CHUNK)],
                sw.at[bi])
            nj = j + _NBUF
            if nj < nch:
                cw[j].wait()
                cg[nj] = pltpu.async_copy(
                    tab_hbm.at[idx_v.at[pl.ds(nj * _CHUNK, _CHUNK)]],
                    rows_v.at[bi], sg.at[bi])
        for j in range(max(nch - _NBUF, 0), nch):
            cw[j].wait()

    return k(idx_flat, table)


_RBLK = 2048
_NBLK = _ROWS // _RBLK


def _mm0_body(xg_ref, cp_ref, w_ref, b_ref, y_ref, s_ref, ss_ref):
    i = pl.program_id(0)

    @pl.when(i == 0)
    def _():
        s_ref[...] = jnp.zeros_like(s_ref)
        ss_ref[...] = jnp.zeros_like(ss_ref)

    c = cp_ref[...]
    cexp = jnp.broadcast_to(c[:, None, :], (_RBLK // _S, _S, _CP))
    x = xg_ref[...] - cexp.reshape(_RBLK, _CP)
    y = jnp.dot(x, w_ref[...], preferred_element_type=jnp.float32,
                precision=_HIGH) + b_ref[...]
    y_ref[...] = y
    s_ref[...] += jnp.sum(y, axis=0, keepdims=True)
    ss_ref[...] += jnp.sum(y * y, axis=0, keepdims=True)


def _mm0(xg, cpad, w0p, b0r):
    return pl.pallas_call(
        _mm0_body,
        grid=(_NBLK,),
        in_specs=[
            pl.BlockSpec((_RBLK, _CP), lambda i: (i, 0)),
            pl.BlockSpec((_RBLK // _S, _CP), lambda i: (i, 0)),
            pl.BlockSpec((_CP, 64), lambda i: (0, 0)),
            pl.BlockSpec((1, 64), lambda i: (0, 0)),
        ],
        out_specs=[
            pl.BlockSpec((_RBLK, 64), lambda i: (i, 0)),
            pl.BlockSpec((1, 64), lambda i: (0, 0)),
            pl.BlockSpec((1, 64), lambda i: (0, 0)),
        ],
        out_shape=[
            jax.ShapeDtypeStruct((_ROWS, 64), jnp.float32),
            jax.ShapeDtypeStruct((1, 64), jnp.float32),
            jax.ShapeDtypeStruct((1, 64), jnp.float32),
        ],
    )(xg, cpad, w0p, b0r)


def _bnmm_body(y_ref, s_ref, ss_ref, g_ref, be_ref, w_ref, b_ref,
               o_ref, so_ref, sso_ref):
    i = pl.program_id(0)

    @pl.when(i == 0)
    def _():
        so_ref[...] = jnp.zeros_like(so_ref)
        sso_ref[...] = jnp.zeros_like(sso_ref)

    inv_n = 1.0 / float(_ROWS)
    mean = s_ref[...] * inv_n
    var = ss_ref[...] * inv_n - mean * mean
    rstd = 1.0 / jnp.sqrt(var + _EPS)
    z = (y_ref[...] - mean) * rstd * g_ref[...] + be_ref[...]
    z = jnp.maximum(z, 0.0)
    o = jnp.dot(z, w_ref[...], preferred_element_type=jnp.float32,
                precision=_HIGH) + b_ref[...]
    o_ref[...] = o
    so_ref[...] += jnp.sum(o, axis=0, keepdims=True)
    sso_ref[...] += jnp.sum(o * o, axis=0, keepdims=True)


def _bnmm(y, s, ss, g, be, wt, br, cout):
    cin = y.shape[1]
    return pl.pallas_call(
        _bnmm_body,
        grid=(_NBLK,),
        in_specs=[
            pl.BlockSpec((_RBLK, cin), lambda i: (i, 0)),
            pl.BlockSpec((1, cin), lambda i: (0, 0)),
            pl.BlockSpec((1, cin), lambda i: (0, 0)),
            pl.BlockSpec((1, cin), lambda i: (0, 0)),
            pl.BlockSpec((1, cin), lambda i: (0, 0)),
            pl.BlockSpec((cin, cout), lambda i: (0, 0)),
            pl.BlockSpec((1, cout), lambda i: (0, 0)),
        ],
        out_specs=[
            pl.BlockSpec((_RBLK, cout), lambda i: (i, 0)),
            pl.BlockSpec((1, cout), lambda i: (0, 0)),
            pl.BlockSpec((1, cout), lambda i: (0, 0)),
        ],
        out_shape=[
            jax.ShapeDtypeStruct((_ROWS, cout), jnp.float32),
            jax.ShapeDtypeStruct((1, cout), jnp.float32),
            jax.ShapeDtypeStruct((1, cout), jnp.float32),
        ],
    )(y, s, ss, g, be, wt, br)


def _bnpool_body(y_ref, s_ref, ss_ref, g_ref, be_ref, o_ref):
    inv_n = 1.0 / float(_ROWS)
    mean = s_ref[...] * inv_n
    var = ss_ref[...] * inv_n - mean * mean
    rstd = 1.0 / jnp.sqrt(var + _EPS)
    z = (y_ref[...] - mean) * rstd * g_ref[...] + be_ref[...]
    z = jnp.maximum(z, 0.0)
    o_ref[...] = jnp.max(z.reshape(_RBLK // _S, _S, 128), axis=1)


def _bnpool(y, s, ss, g, be):
    return pl.pallas_call(
        _bnpool_body,
        grid=(_NBLK,),
        in_specs=[
            pl.BlockSpec((_RBLK, 128), lambda i: (i, 0)),
            pl.BlockSpec((1, 128), lambda i: (0, 0)),
            pl.BlockSpec((1, 128), lambda i: (0, 0)),
            pl.BlockSpec((1, 128), lambda i: (0, 0)),
            pl.BlockSpec((1, 128), lambda i: (0, 0)),
        ],
        out_specs=pl.BlockSpec((_RBLK // _S, 128), lambda i: (i, 0)),
        out_shape=jax.ShapeDtypeStruct((_B * _M, 128), jnp.float32),
    )(y, s, ss, g, be)


def kernel(points, features, W0, b0, gamma0, beta0, W1, b1, gamma1, beta1,
           W2, b2, gamma2, beta2):
    cxyz, t1, t2 = _fps(points)

    table = _prep(points, features)
    c8 = jnp.pad(cxyz, ((0, 0), (0, 5), (0, 0)))
    idx_t = _ballq(table, c8, t1, t2[:, :, None])
    idx = jnp.transpose(idx_t, (0, 2, 1))

    xg = _gather_rows(idx.reshape(_ROWS), table)

    crows = jnp.transpose(cxyz, (0, 2, 1)).reshape(_B * _M, 3)
    cpad = jnp.pad(crows, ((0, 0), (0, _CP - 3)))

    w0p = jnp.pad(W0.T, ((0, _CP - W0.shape[1]), (0, 0)))
    y0, s0, ss0 = _mm0(xg, cpad, w0p, b0.reshape(1, -1))
    y1, s1, ss1 = _bnmm(y0, s0, ss0, gamma0.reshape(1, -1),
                        beta0.reshape(1, -1), W1.T, b1.reshape(1, -1), 64)
    y2, s2, ss2 = _bnmm(y1, s1, ss1, gamma1.reshape(1, -1),
                        beta1.reshape(1, -1), W2.T, b2.reshape(1, -1), 128)
    pooled = _bnpool(y2, s2, ss2, gamma2.reshape(1, -1), beta2.reshape(1, -1))

    new_features = jnp.transpose(pooled.reshape(_B, _M, 128), (0, 2, 1))
    return (cxyz, new_features)
